# baseline (speedup 1.0000x reference)
import functools

import jax
import jax.numpy as jnp
from jax import lax
from jax.experimental import pallas as pl
from jax.experimental.pallas import tpu as pltpu
from jax.experimental.pallas import tpu_sc as plsc

B = 16384
NF = 32
NBIN = 30
NU = 1000000
NI = 100000
NDAY = 366
L = 16
NC = 2
NS = 16
NW = NC * NS
BPW = B // NW
NCH = BPW // L

PU = 1000192
PI = 100352
PK = 512

SPAN = 124928
NFULL = 8 * SPAN
NTAIL = NU - NFULL

_LN2 = 0.6931471805599453
_SQRT2 = 1.4142135623730951
_BETA = 0.4


LCH = 7424
NCHD = 16
LCHR = SPAN - NCHD * LCH


def _detile_body(wpu3, auk3, wpu_t, auk_t,
                 user_r, item_r, tbin_r, tday_r, mc_r, mean_r, bu_r,
                 alpha_r, bcu_r, bi_r, wbit_r, btday_r, wcu_r, gm_r,
                 wpu_o, auk_o, part_o, devt_o,
                 buf_a, buf_b, tbuf,
                 u_v, i_v, tb_v, td_v, mc_v, w_v,
                 bu_v, al_v, me_v, bc_v, bi_v, btd_v, wcu_v, wb_v,
                 gm_v, part_v, devt_v,
                 sem_g, sem_r, sem_w):
    wid = lax.axis_index("s") * NC + lax.axis_index("c")
    i = wid % 4
    span = wid // 4
    c0 = span * SPAN
    base = wid * BPW
    bufs = (buf_a, buf_b)

    pltpu.sync_copy(user_r.at[pl.ds(base, BPW)], u_v)
    pltpu.sync_copy(item_r.at[pl.ds(base, BPW)], i_v)
    pltpu.sync_copy(tbin_r.at[pl.ds(base, BPW)], tb_v)
    pltpu.sync_copy(tday_r.at[pl.ds(base, BPW)], td_v)
    pltpu.sync_copy(mc_r.at[pl.ds(base, BPW)], mc_v)
    pltpu.sync_copy(gm_r, gm_v)

    def widx_body(q, c):
        sl = pl.ds(q * L, L)
        w_v[sl] = tb_v[sl] * PI + i_v[sl]
        return c
    lax.fori_loop(0, NCH, widx_body, 0)

    gathers = [
        pltpu.async_copy(bu_r.at[u_v], bu_v, sem_g),
        pltpu.async_copy(alpha_r.at[u_v], al_v, sem_g),
        pltpu.async_copy(mean_r.at[u_v], me_v, sem_g),
        pltpu.async_copy(bcu_r.at[u_v], bc_v, sem_g),
        pltpu.async_copy(bi_r.at[i_v], bi_v, sem_g),
        pltpu.async_copy(btday_r.at[mc_v], btd_v, sem_g),
        pltpu.async_copy(wcu_r.at[mc_v], wcu_v, sem_g),
        pltpu.async_copy(wbit_r.at[w_v], wb_v, sem_g),
    ]

    chunks = [(c * LCH, LCH) for c in range(NCHD)] + [(NCHD * LCH, LCHR)]
    for src, dst in ((wpu3, wpu_o), (auk3, auk_o)):
        writes = [[], []]
        for c, (coff, clen) in enumerate(chunks):
            b = c % 2
            for h in writes[b]:
                h.wait()
            off = pl.multiple_of(c0 + coff, 128)
            pltpu.async_copy(src.at[i, :, pl.ds(off, clen)],
                             bufs[b].at[:, pl.ds(0, clen)], sem_r).wait()
            writes[b] = [
                pltpu.async_copy(
                    bufs[b].at[r, pl.ds(0, clen)],
                    dst.at[pl.ds(pl.multiple_of((8 * i + r) * PU + off, 8),
                                 clen)],
                    sem_w)
                for r in range(8)]
        for b in (0, 1):
            for h in writes[b]:
                h.wait()

    for w, (t_src, dst) in enumerate(((wpu_t, wpu_o), (auk_t, auk_o))):
        @pl.when(wid == w)
        def _tail(t_src=t_src, dst=dst):
            for g in range(16):
                pltpu.sync_copy(t_src.at[pl.ds(g * 2 * NTAIL, 2 * NTAIL)],
                                tbuf)
                for r in range(2):
                    row = 2 * g + r
                    pltpu.sync_copy(
                        tbuf.at[pl.ds(r * NTAIL, NTAIL)],
                        dst.at[pl.ds(row * PU + NFULL, NTAIL)])

    for h in gathers:
        h.wait()

    gm16 = gm_v[...]

    def bias_body(k, c):
        sl = pl.ds(k * L, L)
        diff = td_v[sl].astype(jnp.float32) - me_v[sl]
        sgn = jnp.sign(diff)
        t = jnp.abs(diff)
        bits = lax.bitcast_convert_type(t, jnp.int32)
        e_i = (bits >> 23) - 127
        m = lax.bitcast_convert_type((bits & 0x7FFFFF) | 0x3F800000,
                                     jnp.float32)
        big = m > _SQRT2
        m = jnp.where(big, m * 0.5, m)
        e_f = e_i.astype(jnp.float32) + jnp.where(big, 1.0, 0.0)
        z = (m - 1.0) / (m + 1.0)
        z2 = z * z
        poly = 1.0 + z2 * ((1.0 / 3.0) + z2 * ((1.0 / 5.0) + z2 * (1.0 / 7.0)))
        ln_t = e_f * _LN2 + 2.0 * z * poly
        devt = sgn * jnp.exp(_BETA * ln_t)
        devt_v[sl] = devt
        part_v[sl] = (gm16 + bu_v[sl] + al_v[sl] * devt + btd_v[sl]
                      + (bi_v[sl] + wb_v[sl]) * (bc_v[sl] + wcu_v[sl]))
        return c

    lax.fori_loop(0, NCH, bias_body, 0)
    pltpu.sync_copy(part_v, part_o.at[pl.ds(base, BPW)])
    pltpu.sync_copy(devt_v, devt_o.at[pl.ds(base, BPW)])


def _main_body(user_r, item_r, mc_r, wpu_r, wpi_r, auk_r, pkut_r,
               part_r, devt_r, out_r,
               u_v, i_v, mc_v, uidx, iidx,
               wpu_c, auk_c, wpi_c, pkut_l,
               part_v, devt_v, out_v, sem):
    wid = lax.axis_index("s") * NC + lax.axis_index("c")
    base = wid * BPW

    pltpu.sync_copy(user_r.at[pl.ds(base, BPW)], u_v)
    pltpu.sync_copy(item_r.at[pl.ds(base, BPW)], i_v)

    def idx_body(q, c):
        sl = pl.ds(q * L, L)
        u16 = u_v[sl]
        i16 = i_v[sl]
        for f in range(NF):
            uidx[pl.ds(f * BPW + q * L, L)] = u16 + f * PU
            iidx[pl.ds(f * BPW + q * L, L)] = i16 + f * PI
        return c
    lax.fori_loop(0, NCH, idx_body, 0)

    copies = [
        pltpu.async_copy(wpu_r.at[uidx], wpu_c, sem),
        pltpu.async_copy(auk_r.at[uidx], auk_c, sem),
        pltpu.async_copy(wpi_r.at[iidx], wpi_c, sem),
    ]
    pltpu.sync_copy(mc_r.at[pl.ds(base, BPW)], mc_v)
    pltpu.sync_copy(part_r.at[pl.ds(base, BPW)], part_v)
    pltpu.sync_copy(devt_r.at[pl.ds(base, BPW)], devt_v)
    pltpu.sync_copy(pkut_r, pkut_l)
    for c in copies:
        c.wait()

    def chunk_body(k, c):
        b16 = k * L
        sl = pl.ds(b16, L)
        devt = devt_v[sl]
        mc16 = mc_v[sl]
        acc = part_v[sl]
        for f in range(NF):
            fsl = pl.ds(f * BPW + b16, L)
            pk16 = plsc.load_gather(pkut_l, [mc16 + f * PK])
            acc = acc + (wpu_c[fsl] + auk_c[fsl] * devt + pk16) * wpi_c[fsl]
        out_v[sl] = acc
        return c

    lax.fori_loop(0, NCH, chunk_body, 0)

    pltpu.sync_copy(out_v, out_r.at[pl.ds(base, BPW)])


@jax.jit
def _run(user, item, tbin, tday, mc, mean_ud, bu, alpha, AlphaUK, bcu,
         WPU, wpi_f, bi, wbit_f, pkut_f, btday, wcu, gm16):
    mesh = plsc.VectorSubcoreMesh(core_axis_name="c", subcore_axis_name="s")

    wpu3 = jnp.transpose(WPU).reshape(4, 8, NU)
    auk3 = jnp.transpose(AlphaUK).reshape(4, 8, NU)
    wpu_t = jnp.transpose(WPU[NFULL:]).reshape(-1)
    auk_t = jnp.transpose(AlphaUK[NFULL:]).reshape(-1)

    detile = functools.partial(
        pl.kernel,
        out_type=(jax.ShapeDtypeStruct((NF * PU,), jnp.float32),
                  jax.ShapeDtypeStruct((NF * PU,), jnp.float32),
                  jax.ShapeDtypeStruct((B,), jnp.float32),
                  jax.ShapeDtypeStruct((B,), jnp.float32)),
        mesh=mesh,
        compiler_params=pltpu.CompilerParams(needs_layout_passes=False,
                                             use_tc_tiling_on_sc=True),
        scratch_types=[
            pltpu.VMEM((8, LCH), jnp.float32),
            pltpu.VMEM((8, LCH), jnp.float32),
            pltpu.VMEM((2 * NTAIL,), jnp.float32),
            pltpu.VMEM((BPW,), jnp.int32),
            pltpu.VMEM((BPW,), jnp.int32),
            pltpu.VMEM((BPW,), jnp.int32),
            pltpu.VMEM((BPW,), jnp.int32),
            pltpu.VMEM((BPW,), jnp.int32),
            pltpu.VMEM((BPW,), jnp.int32),
            pltpu.VMEM((BPW,), jnp.float32),
            pltpu.VMEM((BPW,), jnp.float32),
            pltpu.VMEM((BPW,), jnp.float32),
            pltpu.VMEM((BPW,), jnp.float32),
            pltpu.VMEM((BPW,), jnp.float32),
            pltpu.VMEM((BPW,), jnp.float32),
            pltpu.VMEM((BPW,), jnp.float32),
            pltpu.VMEM((BPW,), jnp.float32),
            pltpu.VMEM((L,), jnp.float32),
            pltpu.VMEM((BPW,), jnp.float32),
            pltpu.VMEM((BPW,), jnp.float32),
            pltpu.SemaphoreType.DMA,
            pltpu.SemaphoreType.DMA,
            pltpu.SemaphoreType.DMA,
        ],
    )(_detile_body)
    wpu_fl, auk_fl, part, devt = detile(
        wpu3, auk3, wpu_t, auk_t, user, item, tbin, tday, mc, mean_ud,
        bu, alpha, bcu, bi, wbit_f, btday, wcu, gm16)

    f = functools.partial(
        pl.kernel,
        out_type=jax.ShapeDtypeStruct((B,), jnp.float32),
        mesh=mesh,
        compiler_params=pltpu.CompilerParams(needs_layout_passes=False,
                                             use_tc_tiling_on_sc=False),
        scratch_types=[
            pltpu.VMEM((BPW,), jnp.int32),
            pltpu.VMEM((BPW,), jnp.int32),
            pltpu.VMEM((BPW,), jnp.int32),
            pltpu.VMEM((NF * BPW,), jnp.int32),
            pltpu.VMEM((NF * BPW,), jnp.int32),
            pltpu.VMEM((NF * BPW,), jnp.float32),
            pltpu.VMEM((NF * BPW,), jnp.float32),
            pltpu.VMEM((NF * BPW,), jnp.float32),
            pltpu.VMEM((NF * PK,), jnp.float32),
            pltpu.VMEM((BPW,), jnp.float32),
            pltpu.VMEM((BPW,), jnp.float32),
            pltpu.VMEM((BPW,), jnp.float32),
            pltpu.SemaphoreType.DMA,
        ],
    )(_main_body)
    return f(user, item, mc, wpu_fl, wpi_f, auk_fl, pkut_f, part, devt)


def _flat(table, pitch):
    t = jnp.transpose(table)
    t = jnp.pad(t, ((0, 0), (0, pitch - t.shape[1])))
    return t.reshape(-1)


def kernel(user, item, tbin, tday, mean_ud, global_mean, maxday_cat,
           WPI, WPU, BU, BI, WBIT, Alpha, AlphaUK, WPUKT, BTDay, BCU, WCU):
    gm16 = jnp.broadcast_to(jnp.float32(global_mean), (L,))
    return _run(user.astype(jnp.int32), item.astype(jnp.int32),
                tbin.astype(jnp.int32), tday.astype(jnp.int32),
                maxday_cat.astype(jnp.int32), mean_ud, BU, Alpha,
                AlphaUK, BCU, WPU, _flat(WPI, PI),
                BI, _flat(WBIT, PI), _flat(WPUKT, PK), BTDay, WCU, gm16)


# speedup vs baseline: 1.0330x; 1.0330x over previous
import functools

import jax
import jax.numpy as jnp
from jax import lax
from jax.experimental import pallas as pl
from jax.experimental.pallas import tpu as pltpu
from jax.experimental.pallas import tpu_sc as plsc

B = 16384
NF = 32
NBIN = 30
NU = 1000000
NI = 100000
NDAY = 366
L = 16
NC = 2
NS = 16
NW = NC * NS
BPW = B // NW
NCH = BPW // L

PU = 1000192
PI = 100352
PK = 512

SPAN = 124928
NFULL = 8 * SPAN
NTAIL = NU - NFULL

_LN2 = 0.6931471805599453
_SQRT2 = 1.4142135623730951
_BETA = 0.4


LCH = 7808
NCHD = SPAN // LCH


def _detile_body(wpu3, auk3, wpu_t, auk_t, wpu_o, auk_o,
                 buf_a, buf_b, tbuf, sem_r, sem_w):
    wid = lax.axis_index("s") * NC + lax.axis_index("c")
    i = wid % 4
    span = wid // 4
    c0 = span * SPAN
    bufs = (buf_a, buf_b)

    for src, dst in ((wpu3, wpu_o), (auk3, auk_o)):
        writes = [[], []]
        for c in range(NCHD):
            b = c % 2
            for h in writes[b]:
                h.wait()
            off = pl.multiple_of(c0 + c * LCH, 128)
            pltpu.async_copy(src.at[i, :, pl.ds(off, LCH)], bufs[b],
                             sem_r).wait()
            writes[b] = [
                pltpu.async_copy(
                    bufs[b].at[r],
                    dst.at[pl.ds(pl.multiple_of((8 * i + r) * PU + off, 8),
                                 LCH)],
                    sem_w)
                for r in range(8)]
        for b in (0, 1):
            for h in writes[b]:
                h.wait()

    for w, (t_src, dst) in enumerate(((wpu_t, wpu_o), (auk_t, auk_o))):
        @pl.when(wid == w)
        def _tail(t_src=t_src, dst=dst):
            for g in range(4):
                pltpu.sync_copy(t_src.at[pl.ds(g * 8 * NTAIL, 8 * NTAIL)],
                                tbuf)
                for r in range(8):
                    row = 8 * g + r
                    pltpu.sync_copy(
                        tbuf.at[pl.ds(r * NTAIL, NTAIL)],
                        dst.at[pl.ds(row * PU + NFULL, NTAIL)])


def _main_body(user_r, item_r, tbin_r, tday_r, mc_r, mean_r, bu_r, alpha_r,
               auk_r, bcu_r, wpu_r, wpi_r, bi_r, wbit_r, pkut_r, btday_r,
               wcu_r, gm_r, out_r,
               u_v, i_v, tb_v, td_v, mc_v, w_v,
               uidx, iidx,
               bu_v, al_v, me_v, bc_v, bi_v, btd_v, wcu_v, wb_v,
               wpu_c, auk_c, wpi_c, pkut_l,
               gm_v, out_v, sem):
    wid = lax.axis_index("s") * NC + lax.axis_index("c")
    base = wid * BPW

    pltpu.sync_copy(user_r.at[pl.ds(base, BPW)], u_v)
    pltpu.sync_copy(item_r.at[pl.ds(base, BPW)], i_v)
    pltpu.sync_copy(tbin_r.at[pl.ds(base, BPW)], tb_v)
    pltpu.sync_copy(tday_r.at[pl.ds(base, BPW)], td_v)
    pltpu.sync_copy(mc_r.at[pl.ds(base, BPW)], mc_v)
    pltpu.sync_copy(gm_r, gm_v)
    pltpu.sync_copy(pkut_r, pkut_l)

    def idx_body(q, c):
        sl = pl.ds(q * L, L)
        u16 = u_v[sl]
        i16 = i_v[sl]
        w_v[sl] = tb_v[sl] * PI + i16
        for f in range(NF):
            uidx[pl.ds(f * BPW + q * L, L)] = u16 + f * PU
            iidx[pl.ds(f * BPW + q * L, L)] = i16 + f * PI
        return c
    lax.fori_loop(0, NCH, idx_body, 0)

    copies = [
        pltpu.async_copy(wpu_r.at[uidx], wpu_c, sem),
        pltpu.async_copy(auk_r.at[uidx], auk_c, sem),
        pltpu.async_copy(wpi_r.at[iidx], wpi_c, sem),
        pltpu.async_copy(wbit_r.at[w_v], wb_v, sem),
        pltpu.async_copy(bu_r.at[u_v], bu_v, sem),
        pltpu.async_copy(alpha_r.at[u_v], al_v, sem),
        pltpu.async_copy(mean_r.at[u_v], me_v, sem),
        pltpu.async_copy(bcu_r.at[u_v], bc_v, sem),
        pltpu.async_copy(bi_r.at[i_v], bi_v, sem),
        pltpu.async_copy(btday_r.at[mc_v], btd_v, sem),
        pltpu.async_copy(wcu_r.at[mc_v], wcu_v, sem),
    ]
    for c in copies:
        c.wait()

    gm16 = gm_v[...]

    def chunk_body(k, c):
        b16 = k * L
        sl = pl.ds(b16, L)
        diff = td_v[sl].astype(jnp.float32) - me_v[sl]
        sgn = jnp.sign(diff)
        t = jnp.abs(diff)
        bits = lax.bitcast_convert_type(t, jnp.int32)
        e_i = (bits >> 23) - 127
        m = lax.bitcast_convert_type((bits & 0x7FFFFF) | 0x3F800000,
                                     jnp.float32)
        big = m > _SQRT2
        m = jnp.where(big, m * 0.5, m)
        e_f = e_i.astype(jnp.float32) + jnp.where(big, 1.0, 0.0)
        z = (m - 1.0) / (m + 1.0)
        z2 = z * z
        poly = 1.0 + z2 * ((1.0 / 3.0) + z2 * ((1.0 / 5.0) + z2 * (1.0 / 7.0)))
        ln_t = e_f * _LN2 + 2.0 * z * poly
        devt = sgn * jnp.exp(_BETA * ln_t)

        wb16 = wb_v[sl]
        mc16 = mc_v[sl]
        acc = (gm16 + bu_v[sl] + al_v[sl] * devt + btd_v[sl]
               + (bi_v[sl] + wb16) * (bc_v[sl] + wcu_v[sl]))
        for f in range(NF):
            fsl = pl.ds(f * BPW + b16, L)
            pk16 = plsc.load_gather(pkut_l, [mc16 + f * PK])
            acc = acc + (wpu_c[fsl] + auk_c[fsl] * devt + pk16) * wpi_c[fsl]
        out_v[sl] = acc
        return c

    lax.fori_loop(0, NCH, chunk_body, 0)

    pltpu.sync_copy(out_v, out_r.at[pl.ds(base, BPW)])


@jax.jit
def _run(user, item, tbin, tday, mc, mean_ud, bu, alpha, AlphaUK, bcu,
         WPU, wpi_f, bi, wbit_f, pkut_f, btday, wcu, gm16):
    mesh = plsc.VectorSubcoreMesh(core_axis_name="c", subcore_axis_name="s")

    wpu3 = jnp.transpose(WPU).reshape(4, 8, NU)
    auk3 = jnp.transpose(AlphaUK).reshape(4, 8, NU)
    wpu_t = jnp.transpose(WPU[NFULL:]).reshape(-1)
    auk_t = jnp.transpose(AlphaUK[NFULL:]).reshape(-1)
    detile = functools.partial(
        pl.kernel,
        out_type=(jax.ShapeDtypeStruct((NF * PU,), jnp.float32),
                  jax.ShapeDtypeStruct((NF * PU,), jnp.float32)),
        mesh=mesh,
        compiler_params=pltpu.CompilerParams(needs_layout_passes=False,
                                             use_tc_tiling_on_sc=True),
        scratch_types=[pltpu.VMEM((8, LCH), jnp.float32),
                       pltpu.VMEM((8, LCH), jnp.float32),
                       pltpu.VMEM((8 * NTAIL,), jnp.float32),
                       pltpu.SemaphoreType.DMA,
                       pltpu.SemaphoreType.DMA],
    )(_detile_body)
    wpu_f, auk_f = detile(wpu3, auk3, wpu_t, auk_t)

    f = functools.partial(
        pl.kernel,
        out_type=jax.ShapeDtypeStruct((B,), jnp.float32),
        mesh=mesh,
        compiler_params=pltpu.CompilerParams(needs_layout_passes=False,
                                             use_tc_tiling_on_sc=False),
        scratch_types=[
            pltpu.VMEM((BPW,), jnp.int32),
            pltpu.VMEM((BPW,), jnp.int32),
            pltpu.VMEM((BPW,), jnp.int32),
            pltpu.VMEM((BPW,), jnp.int32),
            pltpu.VMEM((BPW,), jnp.int32),
            pltpu.VMEM((BPW,), jnp.int32),
            pltpu.VMEM((NF * BPW,), jnp.int32),
            pltpu.VMEM((NF * BPW,), jnp.int32),
            pltpu.VMEM((BPW,), jnp.float32),
            pltpu.VMEM((BPW,), jnp.float32),
            pltpu.VMEM((BPW,), jnp.float32),
            pltpu.VMEM((BPW,), jnp.float32),
            pltpu.VMEM((BPW,), jnp.float32),
            pltpu.VMEM((BPW,), jnp.float32),
            pltpu.VMEM((BPW,), jnp.float32),
            pltpu.VMEM((BPW,), jnp.float32),
            pltpu.VMEM((NF * BPW,), jnp.float32),
            pltpu.VMEM((NF * BPW,), jnp.float32),
            pltpu.VMEM((NF * BPW,), jnp.float32),
            pltpu.VMEM((NF * PK,), jnp.float32),
            pltpu.VMEM((L,), jnp.float32),
            pltpu.VMEM((BPW,), jnp.float32),
            pltpu.SemaphoreType.DMA,
        ],
    )(_main_body)
    return f(user, item, tbin, tday, mc, mean_ud, bu, alpha, auk_f, bcu,
             wpu_f, wpi_f, bi, wbit_f, pkut_f, btday, wcu, gm16)


def _flat(table, pitch):
    t = jnp.transpose(table)
    t = jnp.pad(t, ((0, 0), (0, pitch - t.shape[1])))
    return t.reshape(-1)


def kernel(user, item, tbin, tday, mean_ud, global_mean, maxday_cat,
           WPI, WPU, BU, BI, WBIT, Alpha, AlphaUK, WPUKT, BTDay, BCU, WCU):
    gm16 = jnp.broadcast_to(jnp.float32(global_mean), (L,))
    return _run(user.astype(jnp.int32), item.astype(jnp.int32),
                tbin.astype(jnp.int32), tday.astype(jnp.int32),
                maxday_cat.astype(jnp.int32), mean_ud, BU, Alpha,
                AlphaUK, BCU, WPU, _flat(WPI, PI),
                BI, _flat(WBIT, PI), _flat(WPUKT, PK), BTDay, WCU, gm16)
